# baseline (speedup 1.0000x reference)
import jax
import jax.numpy as jnp
from jax.experimental import pallas as pl

_BB = 256


def _add_kernel(x_ref, pe_ref, o_ref):
    o_ref[...] = x_ref[...] + pe_ref[...]


def kernel(x, pos_emb):
    B, T, D = x.shape
    x2 = x.reshape(B, T * D)
    pe2 = pos_emb.reshape(1, T * D)
    out = pl.pallas_call(
        _add_kernel,
        grid=(B // _BB,),
        in_specs=[
            pl.BlockSpec((_BB, T * D), lambda i: (i, 0)),
            pl.BlockSpec((1, T * D), lambda i: (0, 0)),
        ],
        out_specs=pl.BlockSpec((_BB, T * D), lambda i: (i, 0)),
        out_shape=jax.ShapeDtypeStruct((B, T * D), x.dtype),
    )(x2, pe2)
    return out.reshape(B, T, D)
